# Initial kernel scaffold; baseline (speedup 1.0000x reference)
import jax
import jax.numpy as jnp
from jax.experimental import pallas as pl


def kernel(x, edge_index, W_l, W_r, b):
    raise NotImplementedError("write your pallas kernel here")


# speedup vs baseline: 12.7002x; 12.7002x over previous
import functools

import jax
import jax.numpy as jnp
from jax import lax
from jax.experimental import pallas as pl
from jax.experimental.pallas import tpu as pltpu
from jax.experimental.pallas import tpu_sc as plsc

N_NODES = 10000
N_PAD = 10240
D_FEAT = 128
D_OUT = 7
N_EDGES = 320000

NC, NS = 2, 16
NW = NC * NS
RPT = N_PAD // NS
EPW = N_EDGES // NW
CH = 80
NCHUNK = EPW // CH
BLK = 1024
FBLK = 1000


def _project_kernel(x_ref, w_ref, o_ref):
    yz = jnp.dot(x_ref[...], w_ref[...], preferred_element_type=jnp.float32)
    col = lax.broadcasted_iota(jnp.int32, yz.shape, 1)
    o_ref[...] = jnp.where(col == 7, 1.0, yz)


def _finalize_kernel(a0_ref, a1_ref, yz_ref, b_ref, o_ref):
    a = a0_ref[...] + a1_ref[...]
    deg = jnp.maximum(a[:, 7:8], 1.0)
    mean7 = a[:, :7] / deg
    z7 = yz_ref[:, 8:15]
    o_ref[...] = jnp.maximum(mean7 + z7 + b_ref[0:1, :], 0.0)


def _sc_aggregate(y_hbm, src_hbm, dst_hbm, zeros_hbm, out0_hbm, out1_hbm,
                  src_v, dst_v, rows_v, stage_v, acc_sp, sem):
    cid = lax.axis_index("c")
    sid = lax.axis_index("s")
    wid = sid * NC + cid
    r0 = sid * RPT

    pltpu.sync_copy(zeros_hbm.at[pl.ds(r0, RPT)], stage_v)
    pltpu.sync_copy(stage_v, acc_sp.at[pl.ds(r0, RPT)])
    pltpu.sync_copy(src_hbm.at[wid], src_v)
    pltpu.sync_copy(dst_hbm.at[wid], dst_v)
    plsc.subcore_barrier()

    @pl.loop(0, NCHUNK)
    def _(j):
        pltpu.async_copy(y_hbm.at[src_v.at[j]], rows_v, sem).wait()
        pltpu.sync_copy(rows_v, acc_sp.at[dst_v.at[j]], add=True)

    plsc.subcore_barrier()

    pltpu.sync_copy(acc_sp.at[pl.ds(r0, RPT)], stage_v)

    @pl.when(cid == 0)
    def _():
        pltpu.sync_copy(stage_v, out0_hbm.at[pl.ds(r0, RPT)])

    @pl.when(cid == 1)
    def _():
        pltpu.sync_copy(stage_v, out1_hbm.at[pl.ds(r0, RPT)])


_sc_agg_call = functools.partial(
    pl.kernel,
    out_type=[
        jax.ShapeDtypeStruct((N_PAD, 16), jnp.float32),
        jax.ShapeDtypeStruct((N_PAD, 16), jnp.float32),
    ],
    mesh=plsc.VectorSubcoreMesh(core_axis_name="c", subcore_axis_name="s"),
    compiler_params=pltpu.CompilerParams(use_tc_tiling_on_sc=False),
    scratch_types=[
        pltpu.VMEM((NCHUNK, CH), jnp.int32),
        pltpu.VMEM((NCHUNK, CH), jnp.int32),
        pltpu.VMEM((CH, 16), jnp.float32),
        pltpu.VMEM((RPT, 16), jnp.float32),
        pltpu.VMEM_SHARED((N_PAD, 16), jnp.float32),
        pltpu.SemaphoreType.DMA,
    ],
)(_sc_aggregate)


def kernel(x, edge_index, W_l, W_r, b):
    wc = jnp.zeros((D_FEAT, 16), jnp.float32)
    wc = wc.at[:, 0:7].set(W_l).at[:, 8:15].set(W_r)

    yz = pl.pallas_call(
        _project_kernel,
        grid=(N_PAD // BLK,),
        in_specs=[
            pl.BlockSpec((BLK, D_FEAT), lambda i: (i, 0)),
            pl.BlockSpec((D_FEAT, 16), lambda i: (0, 0)),
        ],
        out_specs=pl.BlockSpec((BLK, 16), lambda i: (i, 0)),
        out_shape=jax.ShapeDtypeStruct((N_PAD, 16), jnp.float32),
    )(x, wc)

    src3 = edge_index[0].reshape(NW, NCHUNK, CH)
    dst3 = edge_index[1].reshape(NW, NCHUNK, CH)
    zeros = jnp.zeros((N_PAD, 16), jnp.float32)

    acc0, acc1 = _sc_agg_call(yz, src3, dst3, zeros)

    b2 = jnp.broadcast_to(b.reshape(1, D_OUT), (8, D_OUT))
    out = pl.pallas_call(
        _finalize_kernel,
        grid=(N_NODES // FBLK,),
        in_specs=[
            pl.BlockSpec((FBLK, 16), lambda i: (i, 0)),
            pl.BlockSpec((FBLK, 16), lambda i: (i, 0)),
            pl.BlockSpec((FBLK, 16), lambda i: (i, 0)),
            pl.BlockSpec((8, D_OUT), lambda i: (0, 0)),
        ],
        out_specs=pl.BlockSpec((FBLK, D_OUT), lambda i: (i, 0)),
        out_shape=jax.ShapeDtypeStruct((N_NODES, D_OUT), jnp.float32),
    )(acc0, acc1, yz, b2)
    return out


# speedup vs baseline: 22.0724x; 1.7380x over previous
import functools

import jax
import jax.numpy as jnp
from jax import lax
from jax.experimental import pallas as pl
from jax.experimental.pallas import tpu as pltpu
from jax.experimental.pallas import tpu_sc as plsc

N_NODES = 10000
N_PAD = 10240
D_FEAT = 128
D_OUT = 7
N_EDGES = 320000

NC, NS = 2, 16
NW = NC * NS
RPT = N_PAD // NS
EPW = N_EDGES // NW
CH = 2000
NCHUNK = EPW // CH
BLK = 1024
FBLK = 1000


def _project_kernel(x_ref, w_ref, o_ref):
    yz = jnp.dot(x_ref[...], w_ref[...], preferred_element_type=jnp.float32)
    col = lax.broadcasted_iota(jnp.int32, yz.shape, 1)
    o_ref[...] = jnp.where(col == 7, 1.0, yz)


def _finalize_kernel(a0_ref, a1_ref, yz_ref, b_ref, o_ref):
    a = a0_ref[...] + a1_ref[...]
    deg = jnp.maximum(a[:, 7:8], 1.0)
    mean7 = a[:, :7] / deg
    z7 = yz_ref[:, 8:15]
    o_ref[...] = jnp.maximum(mean7 + z7 + b_ref[0:1, :], 0.0)


def _sc_aggregate(y_hbm, src_hbm, dst_hbm, zeros_hbm, out0_hbm, out1_hbm,
                  src_v, dst_v, rows_v, stage_v, acc_sp, sem):
    cid = lax.axis_index("c")
    sid = lax.axis_index("s")
    wid = sid * NC + cid
    r0 = sid * RPT

    pltpu.sync_copy(zeros_hbm.at[pl.ds(r0, RPT)], stage_v)
    pltpu.sync_copy(stage_v, acc_sp.at[pl.ds(r0, RPT)])
    pltpu.sync_copy(src_hbm.at[wid], src_v)
    pltpu.sync_copy(dst_hbm.at[wid], dst_v)
    plsc.subcore_barrier()

    @pl.loop(0, NCHUNK)
    def _(j):
        pltpu.async_copy(y_hbm.at[src_v.at[j]], rows_v, sem).wait()
        pltpu.sync_copy(rows_v, acc_sp.at[dst_v.at[j]], add=True)

    plsc.subcore_barrier()

    pltpu.sync_copy(acc_sp.at[pl.ds(r0, RPT)], stage_v)

    @pl.when(cid == 0)
    def _():
        pltpu.sync_copy(stage_v, out0_hbm.at[pl.ds(r0, RPT)])

    @pl.when(cid == 1)
    def _():
        pltpu.sync_copy(stage_v, out1_hbm.at[pl.ds(r0, RPT)])


_sc_agg_call = functools.partial(
    pl.kernel,
    out_type=[
        jax.ShapeDtypeStruct((N_PAD, 16), jnp.float32),
        jax.ShapeDtypeStruct((N_PAD, 16), jnp.float32),
    ],
    mesh=plsc.VectorSubcoreMesh(core_axis_name="c", subcore_axis_name="s"),
    compiler_params=pltpu.CompilerParams(use_tc_tiling_on_sc=False),
    scratch_types=[
        pltpu.VMEM((NCHUNK, CH), jnp.int32),
        pltpu.VMEM((NCHUNK, CH), jnp.int32),
        pltpu.VMEM((CH, 16), jnp.float32),
        pltpu.VMEM((RPT, 16), jnp.float32),
        pltpu.VMEM_SHARED((N_PAD, 16), jnp.float32),
        pltpu.SemaphoreType.DMA,
    ],
)(_sc_aggregate)


def kernel(x, edge_index, W_l, W_r, b):
    wc = jnp.zeros((D_FEAT, 16), jnp.float32)
    wc = wc.at[:, 0:7].set(W_l).at[:, 8:15].set(W_r)

    yz = pl.pallas_call(
        _project_kernel,
        grid=(N_PAD // BLK,),
        in_specs=[
            pl.BlockSpec((BLK, D_FEAT), lambda i: (i, 0)),
            pl.BlockSpec((D_FEAT, 16), lambda i: (0, 0)),
        ],
        out_specs=pl.BlockSpec((BLK, 16), lambda i: (i, 0)),
        out_shape=jax.ShapeDtypeStruct((N_PAD, 16), jnp.float32),
    )(x, wc)

    src3 = edge_index[0].reshape(NW, NCHUNK, CH)
    dst3 = edge_index[1].reshape(NW, NCHUNK, CH)
    zeros = jnp.zeros((N_PAD, 16), jnp.float32)

    acc0, acc1 = _sc_agg_call(yz, src3, dst3, zeros)

    b2 = jnp.broadcast_to(b.reshape(1, D_OUT), (8, D_OUT))
    out = pl.pallas_call(
        _finalize_kernel,
        grid=(N_NODES // FBLK,),
        in_specs=[
            pl.BlockSpec((FBLK, 16), lambda i: (i, 0)),
            pl.BlockSpec((FBLK, 16), lambda i: (i, 0)),
            pl.BlockSpec((FBLK, 16), lambda i: (i, 0)),
            pl.BlockSpec((8, D_OUT), lambda i: (0, 0)),
        ],
        out_specs=pl.BlockSpec((FBLK, D_OUT), lambda i: (i, 0)),
        out_shape=jax.ShapeDtypeStruct((N_NODES, D_OUT), jnp.float32),
    )(acc0, acc1, yz, b2)
    return out


# speedup vs baseline: 24.5364x; 1.1116x over previous
import functools

import jax
import jax.numpy as jnp
from jax import lax
from jax.experimental import pallas as pl
from jax.experimental.pallas import tpu as pltpu
from jax.experimental.pallas import tpu_sc as plsc

N_NODES = 10000
N_PAD = 10240
D_FEAT = 128
D_OUT = 7
N_EDGES = 320000

NC, NS = 2, 16
NW = NC * NS
RPT = N_PAD // NS
EPW = N_EDGES // NW
CH = 1000
NCHUNK = EPW // CH


def _project_kernel(x_ref, wl_ref, wr_ref, b_ref, o_ref):
    z1 = jnp.zeros((D_FEAT, 1), jnp.float32)
    w16 = jnp.concatenate([wl_ref[...], z1, wr_ref[...], z1], axis=1)
    bias = jnp.concatenate(
        [jnp.zeros((1, 8), jnp.float32), b_ref[...], jnp.zeros((1, 1), jnp.float32)],
        axis=1)
    yz = jnp.dot(x_ref[...], w16, preferred_element_type=jnp.float32) + bias
    col = lax.broadcasted_iota(jnp.int32, yz.shape, 1)
    o_ref[...] = jnp.where(col == 7, 1.0, yz)


def _finalize_kernel(a0_ref, a1_ref, yz_ref, o_ref):
    a = a0_ref[...] + a1_ref[...]
    deg = jnp.maximum(a[:, 7:8], 1.0)
    o_ref[...] = jnp.maximum(a[:, :7] / deg + yz_ref[:, 8:15], 0.0)


def _sc_aggregate(y_hbm, src_hbm, dst_hbm, zeros_hbm, out0_hbm, out1_hbm,
                  src_v, dst_v, rows_a, rows_b, stage_v, acc_sp,
                  gsem, ssem_a, ssem_b):
    cid = lax.axis_index("c")
    sid = lax.axis_index("s")
    wid = sid * NC + cid
    r0 = sid * RPT

    pltpu.sync_copy(zeros_hbm, stage_v)
    pltpu.sync_copy(stage_v, acc_sp.at[pl.ds(r0, RPT)])
    pltpu.sync_copy(src_hbm.at[wid], src_v)
    pltpu.sync_copy(dst_hbm.at[wid], dst_v)
    plsc.subcore_barrier()

    pltpu.async_copy(y_hbm.at[src_v.at[0]], rows_a, gsem)

    @pl.loop(0, NCHUNK, step=2)
    def _(i):
        for b, cur, oth, scur, soth in ((0, rows_a, rows_b, ssem_a, ssem_b),
                                        (1, rows_b, rows_a, ssem_b, ssem_a)):
            j = i + b
            pltpu.make_async_copy(y_hbm.at[src_v.at[j]], cur, gsem).wait()
            pltpu.async_copy(cur, acc_sp.at[dst_v.at[j]], scur, add=True)
            if b == 0:
                @pl.when(i > 0)
                def _():
                    pltpu.make_async_copy(oth, acc_sp.at[dst_v.at[j]], soth).wait()
            else:
                pltpu.make_async_copy(oth, acc_sp.at[dst_v.at[j]], soth).wait()

            @pl.when(j + 1 < NCHUNK)
            def _():
                pltpu.async_copy(y_hbm.at[src_v.at[j + 1]], oth, gsem)

    pltpu.make_async_copy(rows_b, acc_sp.at[dst_v.at[0]], ssem_b).wait()
    plsc.subcore_barrier()

    pltpu.sync_copy(acc_sp.at[pl.ds(r0, RPT)], stage_v)

    @pl.when(cid == 0)
    def _():
        pltpu.sync_copy(stage_v, out0_hbm.at[pl.ds(r0, RPT)])

    @pl.when(cid == 1)
    def _():
        pltpu.sync_copy(stage_v, out1_hbm.at[pl.ds(r0, RPT)])


_sc_agg_call = functools.partial(
    pl.kernel,
    out_type=[
        jax.ShapeDtypeStruct((N_PAD, 16), jnp.float32),
        jax.ShapeDtypeStruct((N_PAD, 16), jnp.float32),
    ],
    mesh=plsc.VectorSubcoreMesh(core_axis_name="c", subcore_axis_name="s"),
    compiler_params=pltpu.CompilerParams(use_tc_tiling_on_sc=False),
    scratch_types=[
        pltpu.VMEM((NCHUNK, CH), jnp.int32),
        pltpu.VMEM((NCHUNK, CH), jnp.int32),
        pltpu.VMEM((CH, 16), jnp.float32),
        pltpu.VMEM((CH, 16), jnp.float32),
        pltpu.VMEM((RPT, 16), jnp.float32),
        pltpu.VMEM_SHARED((N_PAD, 16), jnp.float32),
        pltpu.SemaphoreType.DMA,
        pltpu.SemaphoreType.DMA,
        pltpu.SemaphoreType.DMA,
    ],
)(_sc_aggregate)


def kernel(x, edge_index, W_l, W_r, b):
    yz = pl.pallas_call(
        _project_kernel,
        grid=(1,),
        in_specs=[
            pl.BlockSpec((N_PAD, D_FEAT), lambda i: (0, 0)),
            pl.BlockSpec((D_FEAT, D_OUT), lambda i: (0, 0)),
            pl.BlockSpec((D_FEAT, D_OUT), lambda i: (0, 0)),
            pl.BlockSpec((1, D_OUT), lambda i: (0, 0)),
        ],
        out_specs=pl.BlockSpec((N_PAD, 16), lambda i: (0, 0)),
        out_shape=jax.ShapeDtypeStruct((N_PAD, 16), jnp.float32),
    )(x, W_l, W_r, b.reshape(1, D_OUT))

    src3 = edge_index[0].reshape(NW, NCHUNK, CH)
    dst3 = edge_index[1].reshape(NW, NCHUNK, CH)
    zeros = jnp.zeros((RPT, 16), jnp.float32)

    acc0, acc1 = _sc_agg_call(yz, src3, dst3, zeros)

    out = pl.pallas_call(
        _finalize_kernel,
        grid=(1,),
        in_specs=[
            pl.BlockSpec((N_NODES, 16), lambda i: (0, 0)),
            pl.BlockSpec((N_NODES, 16), lambda i: (0, 0)),
            pl.BlockSpec((N_NODES, 16), lambda i: (0, 0)),
        ],
        out_specs=pl.BlockSpec((N_NODES, D_OUT), lambda i: (0, 0)),
        out_shape=jax.ShapeDtypeStruct((N_NODES, D_OUT), jnp.float32),
    )(acc0, acc1, yz)
    return out


# speedup vs baseline: 27.1161x; 1.1051x over previous
import functools

import jax
import jax.numpy as jnp
from jax import lax
from jax.experimental import pallas as pl
from jax.experimental.pallas import tpu as pltpu
from jax.experimental.pallas import tpu_sc as plsc

N_NODES = 10000
N_PAD = 10240
D_FEAT = 128
D_OUT = 7
N_EDGES = 320000

NC, NS = 2, 16
NW = NC * NS
RPT = N_PAD // NS
EPW = N_EDGES // NW
CH = 1000
NCHUNK = EPW // CH


def _project_kernel(x_ref, wl_ref, wr_ref, b_ref, y_ref, z_ref):
    z1 = jnp.zeros((D_FEAT, 1), jnp.float32)
    xb = x_ref[...]
    y = jnp.dot(xb, jnp.concatenate([wl_ref[...], z1], axis=1),
                preferred_element_type=jnp.float32)
    col = lax.broadcasted_iota(jnp.int32, y.shape, 1)
    y_ref[...] = jnp.where(col == 7, 1.0, y)
    bias = jnp.concatenate([b_ref[...], jnp.zeros((1, 1), jnp.float32)], axis=1)
    z_ref[...] = jnp.dot(xb, jnp.concatenate([wr_ref[...], z1], axis=1),
                         preferred_element_type=jnp.float32) + bias


def _finalize_kernel(a0_ref, a1_ref, z_ref, o_ref):
    a = a0_ref[...] + a1_ref[...]
    deg = jnp.maximum(a[:, 7:8], 1.0)
    o_ref[...] = jnp.maximum(a[:, :7] / deg + z_ref[:, :7], 0.0)


def _sc_aggregate(y_hbm, src_hbm, dst_hbm, zeros_hbm, out0_hbm, out1_hbm,
                  src_v, dst_v, rows_a, rows_b, stage_v, y_sp, acc_sp,
                  gsem, ssem_a, ssem_b):
    cid = lax.axis_index("c")
    sid = lax.axis_index("s")
    wid = sid * NC + cid
    r0 = sid * RPT

    pltpu.sync_copy(zeros_hbm, stage_v)
    pltpu.sync_copy(stage_v, acc_sp.at[pl.ds(r0, RPT)])
    pltpu.sync_copy(y_hbm.at[pl.ds(r0, RPT)], stage_v)
    pltpu.sync_copy(stage_v, y_sp.at[pl.ds(r0, RPT)])
    pltpu.sync_copy(src_hbm.at[wid], src_v)
    pltpu.sync_copy(dst_hbm.at[wid], dst_v)
    plsc.subcore_barrier()

    pltpu.async_copy(y_sp.at[src_v.at[0]], rows_a, gsem)

    @pl.loop(0, NCHUNK, step=2)
    def _(i):
        for b, cur, oth, scur, soth in ((0, rows_a, rows_b, ssem_a, ssem_b),
                                        (1, rows_b, rows_a, ssem_b, ssem_a)):
            j = i + b
            pltpu.make_async_copy(y_sp.at[src_v.at[j]], cur, gsem).wait()
            pltpu.async_copy(cur, acc_sp.at[dst_v.at[j]], scur, add=True)
            if b == 0:
                @pl.when(i > 0)
                def _():
                    pltpu.make_async_copy(oth, acc_sp.at[dst_v.at[j]], soth).wait()
            else:
                pltpu.make_async_copy(oth, acc_sp.at[dst_v.at[j]], soth).wait()

            @pl.when(j + 1 < NCHUNK)
            def _():
                pltpu.async_copy(y_sp.at[src_v.at[j + 1]], oth, gsem)

    pltpu.make_async_copy(rows_b, acc_sp.at[dst_v.at[0]], ssem_b).wait()
    plsc.subcore_barrier()

    pltpu.sync_copy(acc_sp.at[pl.ds(r0, RPT)], stage_v)

    @pl.when(cid == 0)
    def _():
        pltpu.sync_copy(stage_v, out0_hbm.at[pl.ds(r0, RPT)])

    @pl.when(cid == 1)
    def _():
        pltpu.sync_copy(stage_v, out1_hbm.at[pl.ds(r0, RPT)])


_sc_agg_call = functools.partial(
    pl.kernel,
    out_type=[
        jax.ShapeDtypeStruct((N_PAD, 8), jnp.float32),
        jax.ShapeDtypeStruct((N_PAD, 8), jnp.float32),
    ],
    mesh=plsc.VectorSubcoreMesh(core_axis_name="c", subcore_axis_name="s"),
    compiler_params=pltpu.CompilerParams(use_tc_tiling_on_sc=False),
    scratch_types=[
        pltpu.VMEM((NCHUNK, CH), jnp.int32),
        pltpu.VMEM((NCHUNK, CH), jnp.int32),
        pltpu.VMEM((CH, 8), jnp.float32),
        pltpu.VMEM((CH, 8), jnp.float32),
        pltpu.VMEM((RPT, 8), jnp.float32),
        pltpu.VMEM_SHARED((N_PAD, 8), jnp.float32),
        pltpu.VMEM_SHARED((N_PAD, 8), jnp.float32),
        pltpu.SemaphoreType.DMA,
        pltpu.SemaphoreType.DMA,
        pltpu.SemaphoreType.DMA,
    ],
)(_sc_aggregate)


def kernel(x, edge_index, W_l, W_r, b):
    y8, z8 = pl.pallas_call(
        _project_kernel,
        grid=(1,),
        in_specs=[
            pl.BlockSpec((N_PAD, D_FEAT), lambda i: (0, 0)),
            pl.BlockSpec((D_FEAT, D_OUT), lambda i: (0, 0)),
            pl.BlockSpec((D_FEAT, D_OUT), lambda i: (0, 0)),
            pl.BlockSpec((1, D_OUT), lambda i: (0, 0)),
        ],
        out_specs=[
            pl.BlockSpec((N_PAD, 8), lambda i: (0, 0)),
            pl.BlockSpec((N_PAD, 8), lambda i: (0, 0)),
        ],
        out_shape=[
            jax.ShapeDtypeStruct((N_PAD, 8), jnp.float32),
            jax.ShapeDtypeStruct((N_PAD, 8), jnp.float32),
        ],
    )(x, W_l, W_r, b.reshape(1, D_OUT))

    src3 = edge_index[0].reshape(NW, NCHUNK, CH)
    dst3 = edge_index[1].reshape(NW, NCHUNK, CH)
    zeros = jnp.zeros((RPT, 8), jnp.float32)

    acc0, acc1 = _sc_agg_call(y8, src3, dst3, zeros)

    out = pl.pallas_call(
        _finalize_kernel,
        grid=(1,),
        in_specs=[
            pl.BlockSpec((N_NODES, 8), lambda i: (0, 0)),
            pl.BlockSpec((N_NODES, 8), lambda i: (0, 0)),
            pl.BlockSpec((N_NODES, 8), lambda i: (0, 0)),
        ],
        out_specs=pl.BlockSpec((N_NODES, D_OUT), lambda i: (0, 0)),
        out_shape=jax.ShapeDtypeStruct((N_NODES, D_OUT), jnp.float32),
    )(acc0, acc1, z8)
    return out


# speedup vs baseline: 30.6843x; 1.1316x over previous
import functools

import jax
import jax.numpy as jnp
from jax import lax
from jax.experimental import pallas as pl
from jax.experimental.pallas import tpu as pltpu
from jax.experimental.pallas import tpu_sc as plsc

N_NODES = 10000
N_PAD = 10240
D_FEAT = 128
D_OUT = 7
N_EDGES = 320000

NC, NS = 2, 16
NW = NC * NS
RPT = N_PAD // NS
EPW = N_EDGES // NW
CH = 1000
NCHUNK = EPW // CH


def _project_kernel(x_ref, wl_ref, wr_ref, b_ref, y_ref, z_ref):
    z1 = jnp.zeros((D_FEAT, 1), jnp.float32)
    xb = x_ref[...]
    y = jnp.dot(xb, jnp.concatenate([wl_ref[...], z1], axis=1),
                preferred_element_type=jnp.float32)
    col = lax.broadcasted_iota(jnp.int32, y.shape, 1)
    y_ref[...] = jnp.where(col == 7, 1.0, y)
    bias = jnp.concatenate([b_ref[...], jnp.zeros((1, 1), jnp.float32)], axis=1)
    z_ref[...] = jnp.dot(xb, jnp.concatenate([wr_ref[...], z1], axis=1),
                         preferred_element_type=jnp.float32) + bias


def _finalize_kernel(a0_ref, a1_ref, z_ref, o_ref):
    a = a0_ref[...] + a1_ref[...]
    deg = jnp.maximum(a[:, 7:8], 1.0)
    o_ref[...] = jnp.maximum(a[:, :7] / deg + z_ref[:, :7], 0.0)


def _sc_aggregate(y_hbm, edges_hbm, zeros_hbm, out0_hbm, out1_hbm,
                  src_v, dst_v, rows_a, rows_b, stage_v, y_sp, acc_sp,
                  gsem, ssem_a, ssem_b):
    cid = lax.axis_index("c")
    sid = lax.axis_index("s")
    wid = sid * NC + cid
    r0 = sid * RPT

    pltpu.sync_copy(zeros_hbm, stage_v)
    pltpu.sync_copy(stage_v, acc_sp.at[pl.ds(r0, RPT)])
    pltpu.sync_copy(y_hbm.at[pl.ds(r0, RPT)], stage_v)
    pltpu.sync_copy(stage_v, y_sp.at[pl.ds(r0, RPT)])
    pltpu.sync_copy(edges_hbm.at[0, wid], src_v)
    pltpu.sync_copy(edges_hbm.at[1, wid], dst_v)
    plsc.subcore_barrier()

    pltpu.async_copy(y_sp.at[src_v.at[0]], rows_a, gsem)

    @pl.loop(0, NCHUNK, step=2)
    def _(i):
        for b, cur, oth, scur, soth in ((0, rows_a, rows_b, ssem_a, ssem_b),
                                        (1, rows_b, rows_a, ssem_b, ssem_a)):
            j = i + b
            pltpu.make_async_copy(y_sp.at[src_v.at[j]], cur, gsem).wait()
            pltpu.async_copy(cur, acc_sp.at[dst_v.at[j]], scur, add=True)
            if b == 0:
                @pl.when(i > 0)
                def _():
                    pltpu.make_async_copy(oth, acc_sp.at[dst_v.at[j]], soth).wait()
            else:
                pltpu.make_async_copy(oth, acc_sp.at[dst_v.at[j]], soth).wait()

            @pl.when(j + 1 < NCHUNK)
            def _():
                pltpu.async_copy(y_sp.at[src_v.at[j + 1]], oth, gsem)

    pltpu.make_async_copy(rows_b, acc_sp.at[dst_v.at[0]], ssem_b).wait()
    plsc.subcore_barrier()

    pltpu.sync_copy(acc_sp.at[pl.ds(r0, RPT)], stage_v)

    @pl.when(cid == 0)
    def _():
        pltpu.sync_copy(stage_v, out0_hbm.at[pl.ds(r0, RPT)])

    @pl.when(cid == 1)
    def _():
        pltpu.sync_copy(stage_v, out1_hbm.at[pl.ds(r0, RPT)])


_sc_agg_call = functools.partial(
    pl.kernel,
    out_type=[
        jax.ShapeDtypeStruct((N_PAD, 8), jnp.float32),
        jax.ShapeDtypeStruct((N_PAD, 8), jnp.float32),
    ],
    mesh=plsc.VectorSubcoreMesh(core_axis_name="c", subcore_axis_name="s"),
    compiler_params=pltpu.CompilerParams(use_tc_tiling_on_sc=False),
    scratch_types=[
        pltpu.VMEM((NCHUNK, CH), jnp.int32),
        pltpu.VMEM((NCHUNK, CH), jnp.int32),
        pltpu.VMEM((CH, 8), jnp.float32),
        pltpu.VMEM((CH, 8), jnp.float32),
        pltpu.VMEM((RPT, 8), jnp.float32),
        pltpu.VMEM_SHARED((N_PAD, 8), jnp.float32),
        pltpu.VMEM_SHARED((N_PAD, 8), jnp.float32),
        pltpu.SemaphoreType.DMA,
        pltpu.SemaphoreType.DMA,
        pltpu.SemaphoreType.DMA,
    ],
)(_sc_aggregate)


def kernel(x, edge_index, W_l, W_r, b):
    y8, z8 = pl.pallas_call(
        _project_kernel,
        grid=(1,),
        in_specs=[
            pl.BlockSpec((N_PAD, D_FEAT), lambda i: (0, 0)),
            pl.BlockSpec((D_FEAT, D_OUT), lambda i: (0, 0)),
            pl.BlockSpec((D_FEAT, D_OUT), lambda i: (0, 0)),
            pl.BlockSpec((1, D_OUT), lambda i: (0, 0)),
        ],
        out_specs=[
            pl.BlockSpec((N_PAD, 8), lambda i: (0, 0)),
            pl.BlockSpec((N_PAD, 8), lambda i: (0, 0)),
        ],
        out_shape=[
            jax.ShapeDtypeStruct((N_PAD, 8), jnp.float32),
            jax.ShapeDtypeStruct((N_PAD, 8), jnp.float32),
        ],
    )(x, W_l, W_r, b.reshape(1, D_OUT))

    edges4 = edge_index.reshape(2, NW, NCHUNK, CH)
    zeros = jnp.zeros((RPT, 8), jnp.float32)

    acc0, acc1 = _sc_agg_call(y8, edges4, zeros)

    out = pl.pallas_call(
        _finalize_kernel,
        grid=(1,),
        in_specs=[
            pl.BlockSpec((N_NODES, 8), lambda i: (0, 0)),
            pl.BlockSpec((N_NODES, 8), lambda i: (0, 0)),
            pl.BlockSpec((N_NODES, 8), lambda i: (0, 0)),
        ],
        out_specs=pl.BlockSpec((N_NODES, D_OUT), lambda i: (0, 0)),
        out_shape=jax.ShapeDtypeStruct((N_NODES, D_OUT), jnp.float32),
    )(acc0, acc1, z8)
    return out


# speedup vs baseline: 31.1603x; 1.0155x over previous
import functools

import jax
import jax.numpy as jnp
from jax import lax
from jax.experimental import pallas as pl
from jax.experimental.pallas import tpu as pltpu
from jax.experimental.pallas import tpu_sc as plsc

N_NODES = 10000
N_PAD = 10240
D_FEAT = 128
D_OUT = 7
N_EDGES = 320000

NC, NS = 2, 16
NW = NC * NS
RPT = N_PAD // NS
EPW = N_EDGES // NW
CH = 5000
NCHUNK = EPW // CH


def _project_kernel(x_ref, wl_ref, wr_ref, b_ref, y_ref, z_ref):
    z1 = jnp.zeros((D_FEAT, 1), jnp.float32)
    xb = x_ref[...]
    y = jnp.dot(xb, jnp.concatenate([wl_ref[...], z1], axis=1),
                preferred_element_type=jnp.float32)
    col = lax.broadcasted_iota(jnp.int32, y.shape, 1)
    y_ref[...] = jnp.where(col == 7, 1.0, y)
    bias = jnp.concatenate([b_ref[...], jnp.zeros((1, 1), jnp.float32)], axis=1)
    z_ref[...] = jnp.dot(xb, jnp.concatenate([wr_ref[...], z1], axis=1),
                         preferred_element_type=jnp.float32) + bias


def _finalize_kernel(a0_ref, a1_ref, z_ref, o_ref):
    a = a0_ref[...] + a1_ref[...]
    deg = jnp.maximum(a[:, 7:8], 1.0)
    o_ref[...] = jnp.maximum(a[:, :7] / deg + z_ref[:, :7], 0.0)


def _sc_aggregate(y_hbm, edges_hbm, zeros_hbm, out0_hbm, out1_hbm,
                  src_v, dst_v, rows_a, rows_b, stage_v, y_sp, acc_sp,
                  gsem, ssem_a, ssem_b):
    cid = lax.axis_index("c")
    sid = lax.axis_index("s")
    wid = sid * NC + cid
    r0 = sid * RPT

    pltpu.sync_copy(zeros_hbm, stage_v)
    pltpu.sync_copy(stage_v, acc_sp.at[pl.ds(r0, RPT)])
    pltpu.sync_copy(y_hbm.at[pl.ds(r0, RPT)], stage_v)
    pltpu.sync_copy(stage_v, y_sp.at[pl.ds(r0, RPT)])
    pltpu.sync_copy(edges_hbm.at[0, wid], src_v)
    pltpu.sync_copy(edges_hbm.at[1, wid], dst_v)
    plsc.subcore_barrier()

    pltpu.async_copy(y_sp.at[src_v.at[0]], rows_a, gsem)

    @pl.loop(0, NCHUNK, step=2)
    def _(i):
        for b, cur, oth, scur, soth in ((0, rows_a, rows_b, ssem_a, ssem_b),
                                        (1, rows_b, rows_a, ssem_b, ssem_a)):
            j = i + b
            pltpu.make_async_copy(y_sp.at[src_v.at[j]], cur, gsem).wait()
            pltpu.async_copy(cur, acc_sp.at[dst_v.at[j]], scur, add=True)
            if b == 0:
                @pl.when(i > 0)
                def _():
                    pltpu.make_async_copy(oth, acc_sp.at[dst_v.at[j]], soth).wait()
            else:
                pltpu.make_async_copy(oth, acc_sp.at[dst_v.at[j]], soth).wait()

            @pl.when(j + 1 < NCHUNK)
            def _():
                pltpu.async_copy(y_sp.at[src_v.at[j + 1]], oth, gsem)

    pltpu.make_async_copy(rows_b, acc_sp.at[dst_v.at[0]], ssem_b).wait()
    plsc.subcore_barrier()

    pltpu.sync_copy(acc_sp.at[pl.ds(r0, RPT)], stage_v)

    @pl.when(cid == 0)
    def _():
        pltpu.sync_copy(stage_v, out0_hbm.at[pl.ds(r0, RPT)])

    @pl.when(cid == 1)
    def _():
        pltpu.sync_copy(stage_v, out1_hbm.at[pl.ds(r0, RPT)])


_sc_agg_call = functools.partial(
    pl.kernel,
    out_type=[
        jax.ShapeDtypeStruct((N_PAD, 8), jnp.float32),
        jax.ShapeDtypeStruct((N_PAD, 8), jnp.float32),
    ],
    mesh=plsc.VectorSubcoreMesh(core_axis_name="c", subcore_axis_name="s"),
    compiler_params=pltpu.CompilerParams(use_tc_tiling_on_sc=False),
    scratch_types=[
        pltpu.VMEM((NCHUNK, CH), jnp.int32),
        pltpu.VMEM((NCHUNK, CH), jnp.int32),
        pltpu.VMEM((CH, 8), jnp.float32),
        pltpu.VMEM((CH, 8), jnp.float32),
        pltpu.VMEM((RPT, 8), jnp.float32),
        pltpu.VMEM_SHARED((N_PAD, 8), jnp.float32),
        pltpu.VMEM_SHARED((N_PAD, 8), jnp.float32),
        pltpu.SemaphoreType.DMA,
        pltpu.SemaphoreType.DMA,
        pltpu.SemaphoreType.DMA,
    ],
)(_sc_aggregate)


def kernel(x, edge_index, W_l, W_r, b):
    y8, z8 = pl.pallas_call(
        _project_kernel,
        grid=(1,),
        in_specs=[
            pl.BlockSpec((N_PAD, D_FEAT), lambda i: (0, 0)),
            pl.BlockSpec((D_FEAT, D_OUT), lambda i: (0, 0)),
            pl.BlockSpec((D_FEAT, D_OUT), lambda i: (0, 0)),
            pl.BlockSpec((1, D_OUT), lambda i: (0, 0)),
        ],
        out_specs=[
            pl.BlockSpec((N_PAD, 8), lambda i: (0, 0)),
            pl.BlockSpec((N_PAD, 8), lambda i: (0, 0)),
        ],
        out_shape=[
            jax.ShapeDtypeStruct((N_PAD, 8), jnp.float32),
            jax.ShapeDtypeStruct((N_PAD, 8), jnp.float32),
        ],
    )(x, W_l, W_r, b.reshape(1, D_OUT))

    edges4 = edge_index.reshape(2, NW, NCHUNK, CH)
    zeros = jnp.zeros((RPT, 8), jnp.float32)

    acc0, acc1 = _sc_agg_call(y8, edges4, zeros)

    out = pl.pallas_call(
        _finalize_kernel,
        grid=(1,),
        in_specs=[
            pl.BlockSpec((N_NODES, 8), lambda i: (0, 0)),
            pl.BlockSpec((N_NODES, 8), lambda i: (0, 0)),
            pl.BlockSpec((N_NODES, 8), lambda i: (0, 0)),
        ],
        out_specs=pl.BlockSpec((N_NODES, D_OUT), lambda i: (0, 0)),
        out_shape=jax.ShapeDtypeStruct((N_NODES, D_OUT), jnp.float32),
    )(acc0, acc1, z8)
    return out


# speedup vs baseline: 31.6533x; 1.0158x over previous
import functools

import jax
import jax.numpy as jnp
from jax import lax
from jax.experimental import pallas as pl
from jax.experimental.pallas import tpu as pltpu
from jax.experimental.pallas import tpu_sc as plsc

N_NODES = 10000
N_PAD = 10240
D_FEAT = 128
D_OUT = 7
N_EDGES = 320000

NC, NS = 2, 16
NW = NC * NS
RPT = N_PAD // NS
EPW = N_EDGES // NW
CH = 5000
NCHUNK = EPW // CH


def _project_kernel(x_ref, wl_ref, wr_ref, b_ref, y_ref, z_ref):
    z1 = jnp.zeros((D_FEAT, 1), jnp.float32)
    xb = x_ref[...]
    y = jnp.dot(xb, jnp.concatenate([wl_ref[...], z1], axis=1),
                preferred_element_type=jnp.float32)
    col = lax.broadcasted_iota(jnp.int32, y.shape, 1)
    y_ref[...] = jnp.where(col == 7, 1.0, y)
    bias = jnp.concatenate([b_ref[...], jnp.zeros((1, 1), jnp.float32)], axis=1)
    z_ref[...] = jnp.dot(xb, jnp.concatenate([wr_ref[...], z1], axis=1),
                         preferred_element_type=jnp.float32) + bias


def _finalize_kernel(acc_ref, z_ref, o_ref):
    a = acc_ref[0] + acc_ref[1]
    deg = jnp.maximum(a[:, 7:8], 1.0)
    o_ref[...] = jnp.maximum(a[:, :7] / deg + z_ref[:, :7], 0.0)


def _sc_aggregate(y_hbm, edges_hbm, zeros_hbm, out_hbm,
                  src_v, dst_v, rows_a, rows_b, stage_v, y_sp, acc_sp,
                  gsem, ssem_a, ssem_b):
    cid = lax.axis_index("c")
    sid = lax.axis_index("s")
    wid = sid * NC + cid
    r0 = sid * RPT

    pltpu.sync_copy(zeros_hbm, stage_v)
    pltpu.sync_copy(stage_v, acc_sp.at[pl.ds(r0, RPT)])
    pltpu.sync_copy(y_hbm.at[pl.ds(r0, RPT)], stage_v)
    pltpu.sync_copy(stage_v, y_sp.at[pl.ds(r0, RPT)])
    pltpu.sync_copy(edges_hbm.at[0, wid], src_v)
    pltpu.sync_copy(edges_hbm.at[1, wid], dst_v)
    plsc.subcore_barrier()

    pltpu.async_copy(y_sp.at[src_v.at[0]], rows_a, gsem)

    @pl.loop(0, NCHUNK, step=2)
    def _(i):
        for b, cur, oth, scur, soth in ((0, rows_a, rows_b, ssem_a, ssem_b),
                                        (1, rows_b, rows_a, ssem_b, ssem_a)):
            j = i + b
            pltpu.make_async_copy(y_sp.at[src_v.at[j]], cur, gsem).wait()
            pltpu.async_copy(cur, acc_sp.at[dst_v.at[j]], scur, add=True)
            if b == 0:
                @pl.when(i > 0)
                def _():
                    pltpu.make_async_copy(oth, acc_sp.at[dst_v.at[j]], soth).wait()
            else:
                pltpu.make_async_copy(oth, acc_sp.at[dst_v.at[j]], soth).wait()

            @pl.when(j + 1 < NCHUNK)
            def _():
                pltpu.async_copy(y_sp.at[src_v.at[j + 1]], oth, gsem)

    pltpu.make_async_copy(rows_b, acc_sp.at[dst_v.at[0]], ssem_b).wait()
    plsc.subcore_barrier()

    pltpu.sync_copy(acc_sp.at[pl.ds(r0, RPT)], stage_v)
    pltpu.sync_copy(stage_v, out_hbm.at[cid, pl.ds(r0, RPT)])


_sc_agg_call = functools.partial(
    pl.kernel,
    out_type=jax.ShapeDtypeStruct((2, N_PAD, 8), jnp.float32),
    mesh=plsc.VectorSubcoreMesh(core_axis_name="c", subcore_axis_name="s"),
    compiler_params=pltpu.CompilerParams(use_tc_tiling_on_sc=False),
    scratch_types=[
        pltpu.VMEM((NCHUNK, CH), jnp.int32),
        pltpu.VMEM((NCHUNK, CH), jnp.int32),
        pltpu.VMEM((CH, 8), jnp.float32),
        pltpu.VMEM((CH, 8), jnp.float32),
        pltpu.VMEM((RPT, 8), jnp.float32),
        pltpu.VMEM_SHARED((N_PAD, 8), jnp.float32),
        pltpu.VMEM_SHARED((N_PAD, 8), jnp.float32),
        pltpu.SemaphoreType.DMA,
        pltpu.SemaphoreType.DMA,
        pltpu.SemaphoreType.DMA,
    ],
)(_sc_aggregate)


def kernel(x, edge_index, W_l, W_r, b):
    y8, z8 = pl.pallas_call(
        _project_kernel,
        grid=(1,),
        in_specs=[
            pl.BlockSpec((N_PAD, D_FEAT), lambda i: (0, 0)),
            pl.BlockSpec((D_FEAT, D_OUT), lambda i: (0, 0)),
            pl.BlockSpec((D_FEAT, D_OUT), lambda i: (0, 0)),
            pl.BlockSpec((1, D_OUT), lambda i: (0, 0)),
        ],
        out_specs=[
            pl.BlockSpec((N_PAD, 8), lambda i: (0, 0)),
            pl.BlockSpec((N_PAD, 8), lambda i: (0, 0)),
        ],
        out_shape=[
            jax.ShapeDtypeStruct((N_PAD, 8), jnp.float32),
            jax.ShapeDtypeStruct((N_PAD, 8), jnp.float32),
        ],
    )(x, W_l, W_r, b.reshape(1, D_OUT))

    edges4 = edge_index.reshape(2, NW, NCHUNK, CH)
    zeros = jnp.zeros((RPT, 8), jnp.float32)

    acc = _sc_agg_call(y8, edges4, zeros)

    out = pl.pallas_call(
        _finalize_kernel,
        grid=(1,),
        in_specs=[
            pl.BlockSpec((2, N_NODES, 8), lambda i: (0, 0, 0)),
            pl.BlockSpec((N_NODES, 8), lambda i: (0, 0)),
        ],
        out_specs=pl.BlockSpec((N_NODES, D_OUT), lambda i: (0, 0)),
        out_shape=jax.ShapeDtypeStruct((N_NODES, D_OUT), jnp.float32),
    )(acc, z8)
    return out
